# Initial kernel scaffold; baseline (speedup 1.0000x reference)
import jax
import jax.numpy as jnp
from jax.experimental import pallas as pl


def kernel(params, x_dep, edge_index_dep, batch_dep, x_cfg, edge_index_cfg, batch_cfg, x_ast, edge_index_ast, batch_ast):
    raise NotImplementedError("write your pallas kernel here")


# speedup vs baseline: 1.8524x; 1.8524x over previous
import functools

import jax
import jax.numpy as jnp
from jax import lax
from jax.experimental import pallas as pl
from jax.experimental.pallas import tpu as pltpu
from jax.experimental.pallas import tpu_sc as plsc

N = 10000
E = 160000
B = 64
VOCAB = 5000
EMB = 20
EMBP = 128
HID = 500
HP = 512
G3 = 3 * HP
LAYERS = 3

MBLK = 1000
NCH = 4
CW = 128
TILES = 16
EPT = E // TILES
NACC = 10240
RPT = NACC // TILES
EK = 80
NBLK = EPT // EK
NPAD = 10240
TPW = NPAD // 32
TBLK = 64
_MESH = plsc.VectorSubcoreMesh(core_axis_name="c", subcore_axis_name="s")


@functools.partial(
    pl.kernel,
    out_type=jax.ShapeDtypeStruct((NPAD, EMBP), jnp.float32),
    mesh=_MESH,
    scratch_types=[
        pltpu.VMEM((TBLK,), jnp.int32),
        pltpu.VMEM((TBLK, EMBP), jnp.float32),
        pltpu.SemaphoreType.DMA,
    ],
)
def _sc_embed(emb_hbm, tok_hbm, out_hbm, idx_v, rows_v, sem):
    c = lax.axis_index("c")
    s = lax.axis_index("s")
    wid = s * 2 + c

    def blk(i, carry):
        base = wid * TPW + i * TBLK
        pltpu.sync_copy(tok_hbm.at[pl.ds(base, TBLK)], idx_v)
        pltpu.async_copy(emb_hbm.at[idx_v], rows_v, sem).wait()
        pltpu.sync_copy(rows_v, out_hbm.at[pl.ds(base, TBLK)])
        return carry

    lax.fori_loop(0, TPW // TBLK, blk, 0)


@functools.partial(
    pl.kernel,
    out_type=jax.ShapeDtypeStruct((NCH, NACC, CW), jnp.float32),
    mesh=_MESH,
    scratch_types=[
        pltpu.VMEM((EK,), jnp.int32),
        pltpu.VMEM((EK,), jnp.int32),
        pltpu.VMEM((EK, CW), jnp.float32),
        pltpu.VMEM_SHARED((NACC, CW), jnp.float32),
        pltpu.SemaphoreType.DMA,
    ],
)
def _sc_scatter(hw_hbm, src_hbm, dst_hbm, zz_hbm, m_hbm,
                idx_s, idx_d, rows_v, acc_sh, sem):
    c = lax.axis_index("c")
    s = lax.axis_index("s")
    ebase = s * EPT
    rbase = s * RPT
    for ci in range(NCH // 2):
        chunk = ci * 2 + c
        pltpu.sync_copy(zz_hbm, acc_sh.at[pl.ds(rbase, RPT)])
        plsc.subcore_barrier()

        def blk(i, carry):
            off = ebase + i * EK
            pltpu.sync_copy(src_hbm.at[pl.ds(off, EK)], idx_s)
            pltpu.sync_copy(dst_hbm.at[pl.ds(off, EK)], idx_d)
            pltpu.async_copy(hw_hbm.at[chunk].at[idx_s], rows_v, sem).wait()
            pltpu.sync_copy(rows_v, acc_sh.at[idx_d], add=True)
            return carry

        lax.fori_loop(0, NBLK, blk, 0)
        plsc.subcore_barrier()
        pltpu.sync_copy(acc_sh.at[pl.ds(rbase, RPT)],
                        m_hbm.at[chunk].at[pl.ds(rbase, RPT)])
        plsc.subcore_barrier()


def _tca_body(h_ref, w_ref, whht_ref, bhh_ref, hw_ref, gh_ref):
    h = h_ref[...]
    hb = h.astype(jnp.bfloat16)
    hw = jnp.dot(hb, w_ref[...].astype(jnp.bfloat16),
                 preferred_element_type=jnp.float32)
    for cidx in range(NCH):
        hw_ref[cidx] = hw[:, cidx * CW:(cidx + 1) * CW]
    gh_ref[...] = jnp.dot(hb, whht_ref[...].astype(jnp.bfloat16),
                          preferred_element_type=jnp.float32) + bhh_ref[...]


_tca = pl.pallas_call(
    _tca_body,
    grid=(N // MBLK,),
    in_specs=[
        pl.BlockSpec((MBLK, HP), lambda i: (i, 0)),
        pl.BlockSpec((HP, HP), lambda i: (0, 0)),
        pl.BlockSpec((HP, G3), lambda i: (0, 0)),
        pl.BlockSpec((1, G3), lambda i: (0, 0)),
    ],
    out_specs=[
        pl.BlockSpec((NCH, MBLK, CW), lambda i: (0, i, 0)),
        pl.BlockSpec((MBLK, G3), lambda i: (i, 0)),
    ],
    out_shape=[
        jax.ShapeDtypeStruct((NCH, N, CW), jnp.float32),
        jax.ShapeDtypeStruct((N, G3), jnp.float32),
    ],
)


def _tcb_body(m_ref, h_ref, gh_ref, wiht_ref, bih_ref, o_ref, *, relu):
    m = jnp.concatenate([m_ref[cidx] for cidx in range(NCH)], axis=1)
    gi = jnp.dot(m.astype(jnp.bfloat16), wiht_ref[...].astype(jnp.bfloat16),
                 preferred_element_type=jnp.float32) + bih_ref[...]
    gh = gh_ref[...]
    h = h_ref[...]
    r = jax.nn.sigmoid(gi[:, :HP] + gh[:, :HP])
    z = jax.nn.sigmoid(gi[:, HP:2 * HP] + gh[:, HP:2 * HP])
    n = jnp.tanh(gi[:, 2 * HP:] + r * gh[:, 2 * HP:])
    hn = (1.0 - z) * n + z * h
    if relu:
        hn = jnp.maximum(hn, 0.0)
    o_ref[...] = hn


def _make_tcb(relu):
    return pl.pallas_call(
        functools.partial(_tcb_body, relu=relu),
        grid=(N // MBLK,),
        in_specs=[
            pl.BlockSpec((NCH, MBLK, CW), lambda i: (0, i, 0)),
            pl.BlockSpec((MBLK, HP), lambda i: (i, 0)),
            pl.BlockSpec((MBLK, G3), lambda i: (i, 0)),
            pl.BlockSpec((HP, G3), lambda i: (0, 0)),
            pl.BlockSpec((1, G3), lambda i: (0, 0)),
        ],
        out_specs=pl.BlockSpec((MBLK, HP), lambda i: (i, 0)),
        out_shape=jax.ShapeDtypeStruct((N, HP), jnp.float32),
    )


_tcb = _make_tcb(False)
_tcb_relu = _make_tcb(True)


def _segmax_body(h_ref, b_ref, o_ref):
    i = pl.program_id(0)
    h = h_ref[...]
    b = b_ref[...]
    rows = []
    for j in range(8):
        mask = b == (i * 8 + j)
        rows.append(jnp.max(jnp.where(mask, h, -jnp.inf),
                            axis=0, keepdims=True))
    o_ref[...] = jnp.concatenate(rows, axis=0)


_segmax = pl.pallas_call(
    _segmax_body,
    grid=(B // 8,),
    in_specs=[
        pl.BlockSpec((N, HP), lambda i: (0, 0)),
        pl.BlockSpec((N, 1), lambda i: (0, 0)),
    ],
    out_specs=pl.BlockSpec((8, HP), lambda i: (i, 0)),
    out_shape=jax.ShapeDtypeStruct((B, HP), jnp.float32),
)


def _convpool_body(win_ref, wm_ref, b_ref, o_ref, *, co):
    y = jnp.dot(win_ref[...].astype(jnp.bfloat16),
                wm_ref[...].astype(jnp.bfloat16),
                preferred_element_type=jnp.float32)
    y = jnp.maximum(y + b_ref[...], 0.0)
    p = y[:, :co]
    for j in range(1, 4):
        p = jnp.maximum(p, y[:, j * co:(j + 1) * co])
    o_ref[...] = p


def _make_convpool(rows, kdim, co, mblk):
    return pl.pallas_call(
        functools.partial(_convpool_body, co=co),
        grid=(rows // mblk,),
        in_specs=[
            pl.BlockSpec((mblk, kdim), lambda i: (i, 0)),
            pl.BlockSpec((kdim, 4 * co), lambda i: (0, 0)),
            pl.BlockSpec((1, 4 * co), lambda i: (0, 0)),
        ],
        out_specs=pl.BlockSpec((mblk, co), lambda i: (i, 0)),
        out_shape=jax.ShapeDtypeStruct((rows, co), jnp.float32),
    )


_cp1 = _make_convpool(B * 374, 20, 20, 1496)
_cp2 = _make_convpool(B * 92, 400, 50, 736)
_cp3 = _make_convpool(B * 22, 1000, 100, 352)


def _lin_body(x_ref, w1_ref, b1_ref, w2_ref, b2_ref, o_ref):
    y = jnp.maximum(jnp.dot(x_ref[...].astype(jnp.bfloat16),
                            w1_ref[...].astype(jnp.bfloat16),
                            preferred_element_type=jnp.float32) + b1_ref[...], 0.0)
    o_ref[...] = jnp.maximum(jnp.dot(y.astype(jnp.bfloat16),
                                     w2_ref[...].astype(jnp.bfloat16),
                                     preferred_element_type=jnp.float32)
                             + b2_ref[...], 0.0)


_lin = pl.pallas_call(
    _lin_body,
    out_shape=jax.ShapeDtypeStruct((B, 4), jnp.float32),
)


def _gate_pad_t(w):
    g = w.T.reshape(HID, 3, HID)
    g = jnp.pad(g, ((0, HP - HID), (0, 0), (0, HP - HID)))
    return g.reshape(HP, G3)


def _gate_pad_b(b):
    return jnp.pad(b.reshape(3, HID), ((0, 0), (0, HP - HID))).reshape(1, G3)


def kernel(params, x_dep, edge_index_dep, batch_dep, x_cfg, edge_index_cfg,
           batch_cfg, x_ast, edge_index_ast, batch_ast):
    p = params
    zeros = jnp.zeros((RPT, CW), jnp.float32)

    def run_branch(tokens, eidx, batch, emb, gp):
        emb_pad = jnp.pad(emb, ((0, 0), (0, EMBP - EMB)))
        w_pad = jnp.pad(gp['weight'],
                        ((0, 0), (0, HP - HID), (0, HP - HID)))
        wiht = _gate_pad_t(gp['w_ih'])
        whht = _gate_pad_t(gp['w_hh'])
        bih = _gate_pad_b(gp['b_ih'])
        bhh = _gate_pad_b(gp['b_hh'])
        tok_pad = jnp.concatenate(
            [tokens, jnp.zeros((NPAD - N,), jnp.int32)])
        x32 = _sc_embed(emb_pad, tok_pad)
        h = jnp.pad(x32[:N, :EMB], ((0, 0), (0, HP - EMB)))
        src = eidx[0]
        dst = eidx[1]
        for li in range(LAYERS):
            hw_ch, gh = _tca(h, w_pad[li], whht, bhh)
            m_ch = _sc_scatter(hw_ch, src, dst, zeros)
            tcb = _tcb_relu if li == LAYERS - 1 else _tcb
            h = tcb(m_ch, h, gh, wiht, bih)
        hb = _segmax(h, batch.reshape(N, 1))
        return hb[:, :HID]

    h1 = run_branch(x_dep, edge_index_dep, batch_dep, p['emb1'], p['ggc1'])
    h2 = run_branch(x_cfg, edge_index_cfg, batch_cfg, p['emb2'], p['ggc2'])
    h3 = run_branch(x_ast, edge_index_ast, batch_ast, p['emb3'], p['ggc3'])
    xcat = jnp.concatenate([h1, h2, h3], axis=1)

    def conv_stage(x3, wmat, bias, cp, lo):
        ci = x3.shape[2]
        cols = [x3[:, (j + k):(j + k) + 4 * lo:4, :]
                for j in range(4) for k in range(5)]
        win = jnp.concatenate(cols, axis=-1).reshape(B * lo, 20 * ci)
        base = jnp.transpose(wmat, (2, 1, 0)).reshape(5 * ci, wmat.shape[0])
        wm = jnp.kron(jnp.eye(4, dtype=jnp.float32), base)
        bm = jnp.tile(bias, (4,))[None, :]
        return cp(win, wm, bm)

    p1 = conv_stage(xcat[:, :, None], p['conv1_w'], p['conv1_b'], _cp1, 374)
    p2 = conv_stage(p1.reshape(B, 374, 20), p['conv2_w'], p['conv2_b'], _cp2, 92)
    p3 = conv_stage(p2.reshape(B, 92, 50), p['conv3_w'], p['conv3_b'], _cp3, 22)
    x4 = p3.reshape(B, 2200)
    l1m = jnp.transpose(p['lin1_w'].reshape(500, 100, 22),
                        (2, 1, 0)).reshape(2200, 500)
    return _lin(x4, l1m, p['lin1_b'][None, :], p['lin2_w'].T,
                p['lin2_b'][None, :])


# speedup vs baseline: 3.7946x; 2.0484x over previous
import functools

import jax
import jax.numpy as jnp
from jax import lax
from jax.experimental import pallas as pl
from jax.experimental.pallas import tpu as pltpu
from jax.experimental.pallas import tpu_sc as plsc

N = 10000
E = 160000
B = 64
VOCAB = 5000
EMB = 20
EMBP = 128
HID = 500
HP = 512
G3 = 3 * HP
LAYERS = 3

MBLK = 1000
NCH = 4
CW = 128
TILES = 16
EPT = E // TILES
NACC = 10240
RPT = NACC // TILES
EK = 80
NBLK = EPT // EK
NPAD = 10240
TPW = NPAD // 32
TBLK = 64
_MESH = plsc.VectorSubcoreMesh(core_axis_name="c", subcore_axis_name="s")


@functools.partial(
    pl.kernel,
    out_type=jax.ShapeDtypeStruct((NPAD, EMBP), jnp.float32),
    mesh=_MESH,
    scratch_types=[
        pltpu.VMEM((TBLK,), jnp.int32),
        pltpu.VMEM((TBLK, EMBP), jnp.float32),
        pltpu.SemaphoreType.DMA,
    ],
)
def _sc_embed(emb_hbm, tok_hbm, out_hbm, idx_v, rows_v, sem):
    c = lax.axis_index("c")
    s = lax.axis_index("s")
    wid = s * 2 + c

    def blk(i, carry):
        base = wid * TPW + i * TBLK
        pltpu.sync_copy(tok_hbm.at[pl.ds(base, TBLK)], idx_v)
        pltpu.async_copy(emb_hbm.at[idx_v], rows_v, sem).wait()
        pltpu.sync_copy(rows_v, out_hbm.at[pl.ds(base, TBLK)])
        return carry

    lax.fori_loop(0, TPW // TBLK, blk, 0)


@functools.partial(
    pl.kernel,
    out_type=jax.ShapeDtypeStruct((NCH, NACC, CW), jnp.float32),
    mesh=_MESH,
    scratch_types=[
        pltpu.VMEM((NBLK, 1, EK), jnp.int32),
        pltpu.VMEM((1, EK), jnp.int32),
        pltpu.VMEM((1, EK), jnp.int32),
        pltpu.VMEM((EK, CW), jnp.float32),
        pltpu.VMEM((EK, CW), jnp.float32),
        pltpu.VMEM_SHARED((NACC, CW), jnp.float32),
        pltpu.SemaphoreType.DMA,
        pltpu.SemaphoreType.DMA,
        pltpu.SemaphoreType.DMA,
        pltpu.SemaphoreType.DMA,
    ],
)
def _sc_scatter(hw_hbm, src_hbm, dst_hbm, zz_hbm, m_hbm,
                sidx, didx0, didx1, rows0, rows1, acc_sh,
                semg0, semg1, semd0, semd1):
    c = lax.axis_index("c")
    s = lax.axis_index("s")
    rbase = s * RPT
    dsts = dst_hbm.at[s]
    pltpu.sync_copy(src_hbm.at[s], sidx)
    for ci in range(NCH // 2):
        chunk = ci * 2 + c
        hwc = hw_hbm.at[chunk]
        pltpu.sync_copy(zz_hbm, acc_sh.at[pl.ds(rbase, RPT)])
        plsc.subcore_barrier()

        pltpu.async_copy(hwc.at[sidx.at[0].at[0]], rows0, semg0)
        pltpu.async_copy(hwc.at[sidx.at[1].at[0]], rows1, semg1)
        pltpu.async_copy(dsts.at[0], didx0, semd0)
        pltpu.async_copy(dsts.at[1], didx1, semd1)

        def stage(i, rows, semg, didx, semd):
            pltpu.make_async_copy(hwc.at[sidx.at[i].at[0]], rows, semg).wait()
            pltpu.make_async_copy(dsts.at[i], didx, semd).wait()
            pltpu.sync_copy(rows, acc_sh.at[didx.at[0]], add=True)

            @pl.when(i + 2 < NBLK)
            def _():
                pltpu.async_copy(hwc.at[sidx.at[i + 2].at[0]], rows, semg)
                pltpu.async_copy(dsts.at[i + 2], didx, semd)

        def pair(g, carry):
            stage(2 * g, rows0, semg0, didx0, semd0)
            stage(2 * g + 1, rows1, semg1, didx1, semd1)
            return carry

        lax.fori_loop(0, NBLK // 2, pair, 0)
        stage(NBLK - 1, rows0, semg0, didx0, semd0)
        plsc.subcore_barrier()
        pltpu.sync_copy(acc_sh.at[pl.ds(rbase, RPT)],
                        m_hbm.at[chunk].at[pl.ds(rbase, RPT)])
        plsc.subcore_barrier()


def _tca_body(h_ref, w_ref, whht_ref, bhh_ref, hw_ref, gh_ref):
    h = h_ref[...]
    hb = h.astype(jnp.bfloat16)
    hw = jnp.dot(hb, w_ref[...].astype(jnp.bfloat16),
                 preferred_element_type=jnp.float32)
    for cidx in range(NCH):
        hw_ref[cidx] = hw[:, cidx * CW:(cidx + 1) * CW]
    gh_ref[...] = jnp.dot(hb, whht_ref[...].astype(jnp.bfloat16),
                          preferred_element_type=jnp.float32) + bhh_ref[...]


_tca = pl.pallas_call(
    _tca_body,
    grid=(N // MBLK,),
    in_specs=[
        pl.BlockSpec((MBLK, HP), lambda i: (i, 0)),
        pl.BlockSpec((HP, HP), lambda i: (0, 0)),
        pl.BlockSpec((HP, G3), lambda i: (0, 0)),
        pl.BlockSpec((1, G3), lambda i: (0, 0)),
    ],
    out_specs=[
        pl.BlockSpec((NCH, MBLK, CW), lambda i: (0, i, 0)),
        pl.BlockSpec((MBLK, G3), lambda i: (i, 0)),
    ],
    out_shape=[
        jax.ShapeDtypeStruct((NCH, N, CW), jnp.float32),
        jax.ShapeDtypeStruct((N, G3), jnp.float32),
    ],
)


def _tcb_body(m_ref, h_ref, gh_ref, wiht_ref, bih_ref, o_ref, *, relu):
    m = jnp.concatenate([m_ref[cidx] for cidx in range(NCH)], axis=1)
    gi = jnp.dot(m.astype(jnp.bfloat16), wiht_ref[...].astype(jnp.bfloat16),
                 preferred_element_type=jnp.float32) + bih_ref[...]
    gh = gh_ref[...]
    h = h_ref[...]
    r = jax.nn.sigmoid(gi[:, :HP] + gh[:, :HP])
    z = jax.nn.sigmoid(gi[:, HP:2 * HP] + gh[:, HP:2 * HP])
    n = jnp.tanh(gi[:, 2 * HP:] + r * gh[:, 2 * HP:])
    hn = (1.0 - z) * n + z * h
    if relu:
        hn = jnp.maximum(hn, 0.0)
    o_ref[...] = hn


def _make_tcb(relu):
    return pl.pallas_call(
        functools.partial(_tcb_body, relu=relu),
        grid=(N // MBLK,),
        in_specs=[
            pl.BlockSpec((NCH, MBLK, CW), lambda i: (0, i, 0)),
            pl.BlockSpec((MBLK, HP), lambda i: (i, 0)),
            pl.BlockSpec((MBLK, G3), lambda i: (i, 0)),
            pl.BlockSpec((HP, G3), lambda i: (0, 0)),
            pl.BlockSpec((1, G3), lambda i: (0, 0)),
        ],
        out_specs=pl.BlockSpec((MBLK, HP), lambda i: (i, 0)),
        out_shape=jax.ShapeDtypeStruct((N, HP), jnp.float32),
    )


_tcb = _make_tcb(False)
_tcb_relu = _make_tcb(True)


def _segmax_body(h_ref, b_ref, o_ref):
    i = pl.program_id(0)
    h = h_ref[...]
    b = b_ref[...]
    rows = []
    for j in range(8):
        mask = b == (i * 8 + j)
        rows.append(jnp.max(jnp.where(mask, h, -jnp.inf),
                            axis=0, keepdims=True))
    o_ref[...] = jnp.concatenate(rows, axis=0)


_segmax = pl.pallas_call(
    _segmax_body,
    grid=(B // 8,),
    in_specs=[
        pl.BlockSpec((N, HP), lambda i: (0, 0)),
        pl.BlockSpec((N, 1), lambda i: (0, 0)),
    ],
    out_specs=pl.BlockSpec((8, HP), lambda i: (i, 0)),
    out_shape=jax.ShapeDtypeStruct((B, HP), jnp.float32),
)


def _convpool_body(win_ref, wm_ref, b_ref, o_ref, *, co):
    y = jnp.dot(win_ref[...].astype(jnp.bfloat16),
                wm_ref[...].astype(jnp.bfloat16),
                preferred_element_type=jnp.float32)
    y = jnp.maximum(y + b_ref[...], 0.0)
    p = y[:, :co]
    for j in range(1, 4):
        p = jnp.maximum(p, y[:, j * co:(j + 1) * co])
    o_ref[...] = p


def _make_convpool(rows, kdim, co, mblk):
    return pl.pallas_call(
        functools.partial(_convpool_body, co=co),
        grid=(rows // mblk,),
        in_specs=[
            pl.BlockSpec((mblk, kdim), lambda i: (i, 0)),
            pl.BlockSpec((kdim, 4 * co), lambda i: (0, 0)),
            pl.BlockSpec((1, 4 * co), lambda i: (0, 0)),
        ],
        out_specs=pl.BlockSpec((mblk, co), lambda i: (i, 0)),
        out_shape=jax.ShapeDtypeStruct((rows, co), jnp.float32),
    )


_cp1 = _make_convpool(B * 374, 20, 20, 1496)
_cp2 = _make_convpool(B * 92, 400, 50, 736)
_cp3 = _make_convpool(B * 22, 1000, 100, 352)


def _lin_body(x_ref, w1_ref, b1_ref, w2_ref, b2_ref, o_ref):
    y = jnp.maximum(jnp.dot(x_ref[...].astype(jnp.bfloat16),
                            w1_ref[...].astype(jnp.bfloat16),
                            preferred_element_type=jnp.float32) + b1_ref[...], 0.0)
    o_ref[...] = jnp.maximum(jnp.dot(y.astype(jnp.bfloat16),
                                     w2_ref[...].astype(jnp.bfloat16),
                                     preferred_element_type=jnp.float32)
                             + b2_ref[...], 0.0)


_lin = pl.pallas_call(
    _lin_body,
    out_shape=jax.ShapeDtypeStruct((B, 4), jnp.float32),
)


def _gate_pad_t(w):
    g = w.T.reshape(HID, 3, HID)
    g = jnp.pad(g, ((0, HP - HID), (0, 0), (0, HP - HID)))
    return g.reshape(HP, G3)


def _gate_pad_b(b):
    return jnp.pad(b.reshape(3, HID), ((0, 0), (0, HP - HID))).reshape(1, G3)


def kernel(params, x_dep, edge_index_dep, batch_dep, x_cfg, edge_index_cfg,
           batch_cfg, x_ast, edge_index_ast, batch_ast):
    p = params
    zeros = jnp.zeros((RPT, CW), jnp.float32)

    def run_branch(tokens, eidx, batch, emb, gp):
        emb_pad = jnp.pad(emb, ((0, 0), (0, EMBP - EMB)))
        w_pad = jnp.pad(gp['weight'],
                        ((0, 0), (0, HP - HID), (0, HP - HID)))
        wiht = _gate_pad_t(gp['w_ih'])
        whht = _gate_pad_t(gp['w_hh'])
        bih = _gate_pad_b(gp['b_ih'])
        bhh = _gate_pad_b(gp['b_hh'])
        tok_pad = jnp.concatenate(
            [tokens, jnp.zeros((NPAD - N,), jnp.int32)])
        x32 = _sc_embed(emb_pad, tok_pad)
        h = jnp.pad(x32[:N, :EMB], ((0, 0), (0, HP - EMB)))
        src = eidx[0].reshape(TILES, NBLK, 1, EK)
        dst = eidx[1].reshape(TILES, NBLK, 1, EK)
        for li in range(LAYERS):
            hw_ch, gh = _tca(h, w_pad[li], whht, bhh)
            m_ch = _sc_scatter(hw_ch, src, dst, zeros)
            tcb = _tcb_relu if li == LAYERS - 1 else _tcb
            h = tcb(m_ch, h, gh, wiht, bih)
        hb = _segmax(h, batch.reshape(N, 1))
        return hb[:, :HID]

    h1 = run_branch(x_dep, edge_index_dep, batch_dep, p['emb1'], p['ggc1'])
    h2 = run_branch(x_cfg, edge_index_cfg, batch_cfg, p['emb2'], p['ggc2'])
    h3 = run_branch(x_ast, edge_index_ast, batch_ast, p['emb3'], p['ggc3'])
    xcat = jnp.concatenate([h1, h2, h3], axis=1)

    def conv_stage(x3, wmat, bias, cp, lo):
        ci = x3.shape[2]
        cols = [x3[:, (j + k):(j + k) + 4 * lo:4, :]
                for j in range(4) for k in range(5)]
        win = jnp.concatenate(cols, axis=-1).reshape(B * lo, 20 * ci)
        base = jnp.transpose(wmat, (2, 1, 0)).reshape(5 * ci, wmat.shape[0])
        wm = jnp.kron(jnp.eye(4, dtype=jnp.float32), base)
        bm = jnp.tile(bias, (4,))[None, :]
        return cp(win, wm, bm)

    p1 = conv_stage(xcat[:, :, None], p['conv1_w'], p['conv1_b'], _cp1, 374)
    p2 = conv_stage(p1.reshape(B, 374, 20), p['conv2_w'], p['conv2_b'], _cp2, 92)
    p3 = conv_stage(p2.reshape(B, 92, 50), p['conv3_w'], p['conv3_b'], _cp3, 22)
    x4 = p3.reshape(B, 2200)
    l1m = jnp.transpose(p['lin1_w'].reshape(500, 100, 22),
                        (2, 1, 0)).reshape(2200, 500)
    return _lin(x4, l1m, p['lin1_b'][None, :], p['lin2_w'].T,
                p['lin2_b'][None, :])
